# baseline (speedup 1.0000x reference)
import functools

import jax
import jax.numpy as jnp
from jax import lax
from jax.experimental import pallas as pl
from jax.experimental.pallas import tpu as pltpu
from jax.experimental.pallas import tpu_sc as plsc

DIM_ = 1024
B_, W_ = 64, 2048
TOPK_K = max(1, int(round(W_ * 0.1)))
BETA = 0.6

SC_ROWS = 32768
TC_ROWS = B_ * W_ - SC_ROWS
TILE_A = 512
DEPTH = 8
ODEPTH = 4
N_TILES = TC_ROWS // TILE_A

N_WORKERS = 32
SC_RPW = SC_ROWS // N_WORKERS
SC_CH = 32
SC_NBUF = 2
SC_NCH = SC_RPW // SC_CH
SC_GROUP = 8


def _matmul_kernel(x_hbm, w_ref, o_hbm, buf, sem, obuf, osem):
    wbf = w_ref[...].astype(jnp.bfloat16)

    def start_copy(i, slot):
        pltpu.make_async_copy(
            x_hbm.at[pl.ds(i * TILE_A, TILE_A), :],
            buf.at[slot], sem.at[slot]).start()

    def out_copy(i, oslot):
        return pltpu.make_async_copy(
            obuf.at[oslot],
            o_hbm.at[pl.ds(i * TILE_A, TILE_A), :], osem.at[oslot])

    for j in range(DEPTH):
        start_copy(j, j)

    def body(i, _):
        slot = jax.lax.rem(i, DEPTH)
        oslot = jax.lax.rem(i, ODEPTH)
        pltpu.make_async_copy(
            x_hbm.at[pl.ds(i * TILE_A, TILE_A), :],
            buf.at[slot], sem.at[slot]).wait()

        @pl.when(i >= ODEPTH)
        def _():
            out_copy(i - ODEPTH, oslot).wait()

        obuf[oslot] = jnp.dot(
            buf[slot].astype(jnp.bfloat16), wbf,
            preferred_element_type=jnp.float32)
        out_copy(i, oslot).start()

        @pl.when(i + DEPTH < N_TILES)
        def _():
            start_copy(i + DEPTH, slot)

        return 0

    jax.lax.fori_loop(0, N_TILES, body, 0)
    for j in range(ODEPTH):
        i = N_TILES - ODEPTH + j
        out_copy(i, i % ODEPTH).wait()


def _sc_matvec(x_hbm, wt_hbm, o_hbm, xbuf, wbuf, obuf, sem, osem):
    wid = lax.axis_index("s") * 2 + lax.axis_index("c")
    base = TC_ROWS + wid * SC_RPW
    pltpu.sync_copy(wt_hbm, wbuf)

    def start_copy(g, slot):
        pltpu.make_async_copy(
            x_hbm.at[pl.ds(base + g * SC_CH, SC_CH), :],
            xbuf.at[slot], sem.at[slot]).start()

    for j in range(SC_NBUF):
        start_copy(j, j)

    def chunk_pair(g2, _):
        for b in range(SC_NBUF):
            g = g2 * SC_NBUF + b
            pltpu.make_async_copy(
                x_hbm.at[pl.ds(base + g * SC_CH, SC_CH), :],
                xbuf.at[b], sem.at[b]).wait()

            @pl.when(g >= SC_NBUF)
            def _():
                pltpu.make_async_copy(
                    obuf.at[b],
                    o_hbm.at[pl.ds(base - TC_ROWS + (g - SC_NBUF) * SC_CH,
                                   SC_CH), :, :],
                    osem.at[b]).wait()

            def group(gi, _):
                roff = gi * SC_GROUP
                zero = jnp.zeros((16,), jnp.float32)
                accs = [[zero, zero] for _ in range(SC_GROUP)]
                for j in range(DIM_ // 16):
                    w0 = wbuf[0, pl.ds(j * 16, 16)]
                    w1 = wbuf[1, pl.ds(j * 16, 16)]
                    for ri in range(SC_GROUP):
                        xv = xbuf[b, roff + ri, pl.ds(j * 16, 16)]
                        accs[ri][0] = accs[ri][0] + xv * w0
                        accs[ri][1] = accs[ri][1] + xv * w1
                for ri in range(SC_GROUP):
                    obuf[b, roff + ri, 0, :] = accs[ri][0]
                    obuf[b, roff + ri, 1, :] = accs[ri][1]
                return 0

            lax.fori_loop(0, SC_CH // SC_GROUP, group, 0)
            pltpu.make_async_copy(
                obuf.at[b],
                o_hbm.at[pl.ds(base - TC_ROWS + g * SC_CH, SC_CH), :, :],
                osem.at[b]).start()

            @pl.when(g + SC_NBUF < SC_NCH)
            def _():
                start_copy(g + SC_NBUF, b)

        return 0

    lax.fori_loop(0, SC_NCH // SC_NBUF, chunk_pair, 0)
    for j in range(SC_NBUF):
        g = SC_NCH - SC_NBUF + j
        pltpu.make_async_copy(
            obuf.at[g % SC_NBUF],
            o_hbm.at[pl.ds(base - TC_ROWS + g * SC_CH, SC_CH), :, :],
            osem.at[g % SC_NBUF]).wait()


def _finalize_kernel(cls_ref, attn_ref, cls_sc_ref, attn_sc_ref,
                     mask_ref, bc_ref, ba_ref,
                     logits_ref, probs_ref, vp_ref, vl_ref, aw_ref):
    mask = mask_ref[...]
    cls_full = jnp.concatenate(
        [cls_ref[...], jnp.sum(cls_sc_ref[...], axis=2)], axis=0)
    attn_full = jnp.concatenate(
        [attn_ref[...], jnp.sum(attn_sc_ref[...], axis=2)], axis=0)
    logits = cls_full + bc_ref[0, 0]
    logits_ref[...] = logits
    probs = jax.nn.sigmoid(logits) * mask
    probs_ref[...] = probs

    k = TOPK_K

    def body(_, carry):
        lo, hi = carry
        mid = 0.5 * (lo + hi)
        cnt = jnp.sum((probs > mid).astype(jnp.float32), axis=1,
                      keepdims=True)
        below = cnt < float(k)
        hi = jnp.where(below, mid, hi)
        lo = jnp.where(below, lo, mid)
        return lo, hi

    lo0 = jnp.zeros((B_, 1), jnp.float32)
    hi0 = jnp.ones((B_, 1), jnp.float32)
    lo, hi = jax.lax.fori_loop(0, 46, body, (lo0, hi0))
    t = lo
    gt = probs > t
    cnt_gt = jnp.sum(gt.astype(jnp.float32), axis=1, keepdims=True)
    sum_gt = jnp.sum(jnp.where(gt, probs, 0.0), axis=1, keepdims=True)
    topk_score = (sum_gt + (float(k) - cnt_gt) * t) * (1.0 / float(k))

    alog = attn_full + ba_ref[0, 0]
    alog = jnp.where(mask == 0.0, -10000.0, alog)
    m = jnp.max(alog, axis=1, keepdims=True)
    e = jnp.exp(alog - m)
    s = jnp.sum(e, axis=1, keepdims=True)
    aw = e / s
    aw_ref[...] = aw
    attn_score = jnp.sum(aw * probs, axis=1, keepdims=True)

    video_prob = BETA * topk_score + (1.0 - BETA) * attn_score
    vp_ref[...] = video_prob
    p = jnp.clip(video_prob, 1e-6, 1.0 - 1e-6)
    vl_ref[...] = jnp.log(p / (1.0 - p))


def kernel(window_feat, window_mask, W_cls, b_cls, W_attn, b_attn):
    feat2d = window_feat.reshape(B_ * W_, DIM_)
    wcat = jnp.concatenate([W_cls, W_attn], axis=1)

    sc_mesh = plsc.VectorSubcoreMesh(core_axis_name="c",
                                     subcore_axis_name="s")
    sc_fn = functools.partial(
        pl.kernel, mesh=sc_mesh,
        out_type=jax.ShapeDtypeStruct((SC_ROWS, 2, 16), jnp.float32),
        scratch_types=[
            pltpu.VMEM((SC_NBUF, SC_CH, DIM_), jnp.float32),
            pltpu.VMEM((2, DIM_), jnp.float32),
            pltpu.VMEM((SC_NBUF, SC_CH, 2, 16), jnp.float32),
            pltpu.SemaphoreType.DMA((SC_NBUF,)),
            pltpu.SemaphoreType.DMA((SC_NBUF,)),
        ],
    )(_sc_matvec)
    logits2_sc = sc_fn(feat2d, wcat.T)

    logits2_tc = pl.pallas_call(
        _matmul_kernel,
        in_specs=[
            pl.BlockSpec(memory_space=pltpu.MemorySpace.HBM),
            pl.BlockSpec((DIM_, 2), lambda: (0, 0)),
        ],
        out_specs=pl.BlockSpec(memory_space=pltpu.MemorySpace.HBM),
        out_shape=jax.ShapeDtypeStruct((TC_ROWS, 2), jnp.float32),
        scratch_shapes=[
            pltpu.VMEM((DEPTH, TILE_A, DIM_), jnp.float32),
            pltpu.SemaphoreType.DMA((DEPTH,)),
            pltpu.VMEM((ODEPTH, TILE_A, 2), jnp.float32),
            pltpu.SemaphoreType.DMA((ODEPTH,)),
        ],
    )(feat2d, wcat)

    cls_tc = logits2_tc[:, 0].reshape(TC_ROWS // W_, W_)
    attn_tc = logits2_tc[:, 1].reshape(TC_ROWS // W_, W_)
    cls_sc = logits2_sc[:, 0, :].reshape(SC_ROWS // W_, W_, 16)
    attn_sc = logits2_sc[:, 1, :].reshape(SC_ROWS // W_, W_, 16)
    mask = window_mask.astype(jnp.float32)
    bc = b_cls.reshape(1, 1)
    ba = b_attn.reshape(1, 1)

    outs = pl.pallas_call(
        _finalize_kernel,
        out_shape=[
            jax.ShapeDtypeStruct((B_, W_), jnp.float32),
            jax.ShapeDtypeStruct((B_, W_), jnp.float32),
            jax.ShapeDtypeStruct((B_, 1), jnp.float32),
            jax.ShapeDtypeStruct((B_, 1), jnp.float32),
            jax.ShapeDtypeStruct((B_, W_), jnp.float32),
        ],
    )(cls_tc, attn_tc, cls_sc, attn_sc, mask, bc, ba)

    logits, probs, vp, vl, aw = outs
    return (logits, probs, vp.reshape(B_), vl.reshape(B_), aw)


# speedup vs baseline: 1.1277x; 1.1277x over previous
import functools

import jax
import jax.numpy as jnp
from jax import lax
from jax.experimental import pallas as pl
from jax.experimental.pallas import tpu as pltpu
from jax.experimental.pallas import tpu_sc as plsc

DIM_ = 1024
B_, W_ = 64, 2048
TOPK_K = max(1, int(round(W_ * 0.1)))
BETA = 0.6

SC_ROWS = 8192
TC_ROWS = B_ * W_ - SC_ROWS
TILE_A = 512
DEPTH = 8
ODEPTH = 4
N_TILES = TC_ROWS // TILE_A

N_WORKERS = 32
SC_RPW = SC_ROWS // N_WORKERS
SC_CH = 32
SC_NBUF = 2
SC_NCH = SC_RPW // SC_CH
SC_GROUP = 8


def _matmul_kernel(x_hbm, w_ref, o_hbm, buf, sem, obuf, osem):
    wbf = w_ref[...].astype(jnp.bfloat16)

    def start_copy(i, slot):
        pltpu.make_async_copy(
            x_hbm.at[pl.ds(i * TILE_A, TILE_A), :],
            buf.at[slot], sem.at[slot]).start()

    def out_copy(i, oslot):
        return pltpu.make_async_copy(
            obuf.at[oslot],
            o_hbm.at[pl.ds(i * TILE_A, TILE_A), :], osem.at[oslot])

    for j in range(DEPTH):
        start_copy(j, j)

    def body(i, _):
        slot = jax.lax.rem(i, DEPTH)
        oslot = jax.lax.rem(i, ODEPTH)
        pltpu.make_async_copy(
            x_hbm.at[pl.ds(i * TILE_A, TILE_A), :],
            buf.at[slot], sem.at[slot]).wait()

        @pl.when(i >= ODEPTH)
        def _():
            out_copy(i - ODEPTH, oslot).wait()

        obuf[oslot] = jnp.dot(
            buf[slot].astype(jnp.bfloat16), wbf,
            preferred_element_type=jnp.float32)
        out_copy(i, oslot).start()

        @pl.when(i + DEPTH < N_TILES)
        def _():
            start_copy(i + DEPTH, slot)

        return 0

    jax.lax.fori_loop(0, N_TILES, body, 0)
    for j in range(ODEPTH):
        i = N_TILES - ODEPTH + j
        out_copy(i, i % ODEPTH).wait()


def _sc_matvec(x_hbm, wt_hbm, o_hbm, xbuf, wbuf, obuf, sem, osem):
    wid = lax.axis_index("s") * 2 + lax.axis_index("c")
    base = TC_ROWS + wid * SC_RPW
    pltpu.sync_copy(wt_hbm, wbuf)

    def start_copy(g, slot):
        pltpu.make_async_copy(
            x_hbm.at[pl.ds(base + g * SC_CH, SC_CH), :],
            xbuf.at[slot], sem.at[slot]).start()

    for j in range(SC_NBUF):
        start_copy(j, j)

    def chunk_pair(g2, _):
        for b in range(SC_NBUF):
            g = g2 * SC_NBUF + b
            pltpu.make_async_copy(
                x_hbm.at[pl.ds(base + g * SC_CH, SC_CH), :],
                xbuf.at[b], sem.at[b]).wait()

            @pl.when(g >= SC_NBUF)
            def _():
                pltpu.make_async_copy(
                    obuf.at[b],
                    o_hbm.at[pl.ds(base - TC_ROWS + (g - SC_NBUF) * SC_CH,
                                   SC_CH), :, :],
                    osem.at[b]).wait()

            def group(gi, _):
                roff = gi * SC_GROUP
                zero = jnp.zeros((16,), jnp.float32)
                accs = [[zero, zero] for _ in range(SC_GROUP)]
                for j in range(DIM_ // 16):
                    w0 = wbuf[0, pl.ds(j * 16, 16)]
                    w1 = wbuf[1, pl.ds(j * 16, 16)]
                    for ri in range(SC_GROUP):
                        xv = xbuf[b, roff + ri, pl.ds(j * 16, 16)]
                        accs[ri][0] = accs[ri][0] + xv * w0
                        accs[ri][1] = accs[ri][1] + xv * w1
                for ri in range(SC_GROUP):
                    obuf[b, roff + ri, 0, :] = accs[ri][0]
                    obuf[b, roff + ri, 1, :] = accs[ri][1]
                return 0

            lax.fori_loop(0, SC_CH // SC_GROUP, group, 0)
            pltpu.make_async_copy(
                obuf.at[b],
                o_hbm.at[pl.ds(base - TC_ROWS + g * SC_CH, SC_CH), :, :],
                osem.at[b]).start()

            @pl.when(g + SC_NBUF < SC_NCH)
            def _():
                start_copy(g + SC_NBUF, b)

        return 0

    lax.fori_loop(0, SC_NCH // SC_NBUF, chunk_pair, 0)
    for j in range(SC_NBUF):
        g = SC_NCH - SC_NBUF + j
        pltpu.make_async_copy(
            obuf.at[g % SC_NBUF],
            o_hbm.at[pl.ds(base - TC_ROWS + g * SC_CH, SC_CH), :, :],
            osem.at[g % SC_NBUF]).wait()


def _finalize_kernel(cls_ref, attn_ref, cls_sc_ref, attn_sc_ref,
                     mask_ref, bc_ref, ba_ref,
                     logits_ref, probs_ref, vp_ref, vl_ref, aw_ref):
    mask = mask_ref[...]
    cls_full = jnp.concatenate(
        [cls_ref[...], jnp.sum(cls_sc_ref[...], axis=2)], axis=0)
    attn_full = jnp.concatenate(
        [attn_ref[...], jnp.sum(attn_sc_ref[...], axis=2)], axis=0)
    logits = cls_full + bc_ref[0, 0]
    logits_ref[...] = logits
    probs = jax.nn.sigmoid(logits) * mask
    probs_ref[...] = probs

    k = TOPK_K

    def body(_, carry):
        lo, hi = carry
        mid = 0.5 * (lo + hi)
        cnt = jnp.sum((probs > mid).astype(jnp.float32), axis=1,
                      keepdims=True)
        below = cnt < float(k)
        hi = jnp.where(below, mid, hi)
        lo = jnp.where(below, lo, mid)
        return lo, hi

    lo0 = jnp.zeros((B_, 1), jnp.float32)
    hi0 = jnp.ones((B_, 1), jnp.float32)
    lo, hi = jax.lax.fori_loop(0, 46, body, (lo0, hi0))
    t = lo
    gt = probs > t
    cnt_gt = jnp.sum(gt.astype(jnp.float32), axis=1, keepdims=True)
    sum_gt = jnp.sum(jnp.where(gt, probs, 0.0), axis=1, keepdims=True)
    topk_score = (sum_gt + (float(k) - cnt_gt) * t) * (1.0 / float(k))

    alog = attn_full + ba_ref[0, 0]
    alog = jnp.where(mask == 0.0, -10000.0, alog)
    m = jnp.max(alog, axis=1, keepdims=True)
    e = jnp.exp(alog - m)
    s = jnp.sum(e, axis=1, keepdims=True)
    aw = e / s
    aw_ref[...] = aw
    attn_score = jnp.sum(aw * probs, axis=1, keepdims=True)

    video_prob = BETA * topk_score + (1.0 - BETA) * attn_score
    vp_ref[...] = video_prob
    p = jnp.clip(video_prob, 1e-6, 1.0 - 1e-6)
    vl_ref[...] = jnp.log(p / (1.0 - p))


def kernel(window_feat, window_mask, W_cls, b_cls, W_attn, b_attn):
    feat2d = window_feat.reshape(B_ * W_, DIM_)
    wcat = jnp.concatenate([W_cls, W_attn], axis=1)

    sc_mesh = plsc.VectorSubcoreMesh(core_axis_name="c",
                                     subcore_axis_name="s")
    sc_fn = functools.partial(
        pl.kernel, mesh=sc_mesh,
        out_type=jax.ShapeDtypeStruct((SC_ROWS, 2, 16), jnp.float32),
        scratch_types=[
            pltpu.VMEM((SC_NBUF, SC_CH, DIM_), jnp.float32),
            pltpu.VMEM((2, DIM_), jnp.float32),
            pltpu.VMEM((SC_NBUF, SC_CH, 2, 16), jnp.float32),
            pltpu.SemaphoreType.DMA((SC_NBUF,)),
            pltpu.SemaphoreType.DMA((SC_NBUF,)),
        ],
    )(_sc_matvec)
    logits2_sc = sc_fn(feat2d, wcat.T)

    logits2_tc = pl.pallas_call(
        _matmul_kernel,
        in_specs=[
            pl.BlockSpec(memory_space=pltpu.MemorySpace.HBM),
            pl.BlockSpec((DIM_, 2), lambda: (0, 0)),
        ],
        out_specs=pl.BlockSpec(memory_space=pltpu.MemorySpace.HBM),
        out_shape=jax.ShapeDtypeStruct((TC_ROWS, 2), jnp.float32),
        scratch_shapes=[
            pltpu.VMEM((DEPTH, TILE_A, DIM_), jnp.float32),
            pltpu.SemaphoreType.DMA((DEPTH,)),
            pltpu.VMEM((ODEPTH, TILE_A, 2), jnp.float32),
            pltpu.SemaphoreType.DMA((ODEPTH,)),
        ],
    )(feat2d, wcat)

    cls_tc = logits2_tc[:, 0].reshape(TC_ROWS // W_, W_)
    attn_tc = logits2_tc[:, 1].reshape(TC_ROWS // W_, W_)
    cls_sc = logits2_sc[:, 0, :].reshape(SC_ROWS // W_, W_, 16)
    attn_sc = logits2_sc[:, 1, :].reshape(SC_ROWS // W_, W_, 16)
    mask = window_mask.astype(jnp.float32)
    bc = b_cls.reshape(1, 1)
    ba = b_attn.reshape(1, 1)

    outs = pl.pallas_call(
        _finalize_kernel,
        out_shape=[
            jax.ShapeDtypeStruct((B_, W_), jnp.float32),
            jax.ShapeDtypeStruct((B_, W_), jnp.float32),
            jax.ShapeDtypeStruct((B_, 1), jnp.float32),
            jax.ShapeDtypeStruct((B_, 1), jnp.float32),
            jax.ShapeDtypeStruct((B_, W_), jnp.float32),
        ],
    )(cls_tc, attn_tc, cls_sc, attn_sc, mask, bc, ba)

    logits, probs, vp, vl, aw = outs
    return (logits, probs, vp.reshape(B_), vl.reshape(B_), aw)


# speedup vs baseline: 1.3373x; 1.1858x over previous
import jax
import jax.numpy as jnp
from jax.experimental import pallas as pl
from jax.experimental.pallas import tpu as pltpu

DIM_ = 1024
B_, W_ = 64, 2048
TOPK_K = max(1, int(round(W_ * 0.1)))
BETA = 0.6
TILE_A = 512
DEPTH = 16
N_TILES = (B_ * W_) // TILE_A


ODEPTH = 4


def _matmul_kernel(x_hbm, w_ref, o_hbm, buf, sem, obuf, osem):
    wbf = w_ref[...].astype(jnp.bfloat16)

    def start_copy(i, slot):
        pltpu.make_async_copy(
            x_hbm.at[pl.ds(i * TILE_A, TILE_A), :],
            buf.at[slot], sem.at[slot]).start()

    def out_copy(i, oslot):
        return pltpu.make_async_copy(
            obuf.at[oslot],
            o_hbm.at[pl.ds(i * TILE_A, TILE_A), :], osem.at[oslot])

    for j in range(DEPTH):
        start_copy(j, j)

    def body(i, _):
        slot = jax.lax.rem(i, DEPTH)
        oslot = jax.lax.rem(i, ODEPTH)
        pltpu.make_async_copy(
            x_hbm.at[pl.ds(i * TILE_A, TILE_A), :],
            buf.at[slot], sem.at[slot]).wait()

        @pl.when(i >= ODEPTH)
        def _():
            out_copy(i - ODEPTH, oslot).wait()

        obuf[oslot] = jnp.dot(
            buf[slot].astype(jnp.bfloat16), wbf,
            preferred_element_type=jnp.float32)
        out_copy(i, oslot).start()

        @pl.when(i + DEPTH < N_TILES)
        def _():
            start_copy(i + DEPTH, slot)

        return 0

    jax.lax.fori_loop(0, N_TILES, body, 0)
    for j in range(ODEPTH):
        i = N_TILES - ODEPTH + j
        out_copy(i, i % ODEPTH).wait()


def _finalize_kernel(cls_ref, attn_ref, mask_ref, bc_ref, ba_ref,
                     logits_ref, probs_ref, vp_ref, vl_ref, aw_ref):
    mask = mask_ref[...]
    logits = cls_ref[...] + bc_ref[0, 0]
    logits_ref[...] = logits
    probs = jax.nn.sigmoid(logits) * mask
    probs_ref[...] = probs

    k = TOPK_K

    def body(_, carry):
        lo, hi = carry
        mid = 0.5 * (lo + hi)
        cnt = jnp.sum((probs > mid).astype(jnp.float32), axis=1,
                      keepdims=True)
        below = cnt < float(k)
        hi = jnp.where(below, mid, hi)
        lo = jnp.where(below, lo, mid)
        return lo, hi

    lo0 = jnp.zeros((B_, 1), jnp.float32)
    hi0 = jnp.ones((B_, 1), jnp.float32)
    lo, hi = jax.lax.fori_loop(0, 30, body, (lo0, hi0))
    t = lo
    gt = probs > t
    cnt_gt = jnp.sum(gt.astype(jnp.float32), axis=1, keepdims=True)
    sum_gt = jnp.sum(jnp.where(gt, probs, 0.0), axis=1, keepdims=True)
    topk_score = (sum_gt + (float(k) - cnt_gt) * t) * (1.0 / float(k))

    alog = attn_ref[...] + ba_ref[0, 0]
    alog = jnp.where(mask == 0.0, -10000.0, alog)
    m = jnp.max(alog, axis=1, keepdims=True)
    e = jnp.exp(alog - m)
    s = jnp.sum(e, axis=1, keepdims=True)
    aw = e / s
    aw_ref[...] = aw
    attn_score = jnp.sum(aw * probs, axis=1, keepdims=True)

    video_prob = BETA * topk_score + (1.0 - BETA) * attn_score
    vp_ref[...] = video_prob
    p = jnp.clip(video_prob, 1e-6, 1.0 - 1e-6)
    vl_ref[...] = jnp.log(p / (1.0 - p))


def kernel(window_feat, window_mask, W_cls, b_cls, W_attn, b_attn):
    feat2d = window_feat.reshape(B_ * W_, DIM_)
    wcat = jnp.concatenate([W_cls, W_attn], axis=1)

    logits2 = pl.pallas_call(
        _matmul_kernel,
        in_specs=[
            pl.BlockSpec(memory_space=pltpu.MemorySpace.HBM),
            pl.BlockSpec((DIM_, 2), lambda: (0, 0)),
        ],
        out_specs=pl.BlockSpec(memory_space=pltpu.MemorySpace.HBM),
        out_shape=jax.ShapeDtypeStruct((B_ * W_, 2), jnp.float32),
        scratch_shapes=[
            pltpu.VMEM((DEPTH, TILE_A, DIM_), jnp.float32),
            pltpu.SemaphoreType.DMA((DEPTH,)),
            pltpu.VMEM((ODEPTH, TILE_A, 2), jnp.float32),
            pltpu.SemaphoreType.DMA((ODEPTH,)),
        ],
    )(feat2d, wcat)

    cls_l = logits2[:, 0].reshape(B_, W_)
    attn_l = logits2[:, 1].reshape(B_, W_)
    mask = window_mask.astype(jnp.float32)
    bc = b_cls.reshape(1, 1)
    ba = b_attn.reshape(1, 1)

    outs = pl.pallas_call(
        _finalize_kernel,
        out_shape=[
            jax.ShapeDtypeStruct((B_, W_), jnp.float32),
            jax.ShapeDtypeStruct((B_, W_), jnp.float32),
            jax.ShapeDtypeStruct((B_, 1), jnp.float32),
            jax.ShapeDtypeStruct((B_, 1), jnp.float32),
            jax.ShapeDtypeStruct((B_, W_), jnp.float32),
        ],
    )(cls_l, attn_l, mask, bc, ba)

    logits, probs, vp, vl, aw = outs
    return (logits, probs, vp.reshape(B_), vl.reshape(B_), aw)
